# Initial kernel scaffold; baseline (speedup 1.0000x reference)
import jax
import jax.numpy as jnp
from jax.experimental import pallas as pl


def kernel(image_features, pos_table, masks):
    raise NotImplementedError("write your pallas kernel here")


# speedup vs baseline: 1.1174x; 1.1174x over previous
import jax
import jax.numpy as jnp
from jax.experimental import pallas as pl

M, P, D = 16, 1024, 768


def _body(mask_ref, feat_ref, pos_ref, out_ref):
    feats = feat_ref[...] + pos_ref[...]
    mrow = mask_ref[0, 0, :]
    out_ref[0] = feats * mrow[:, None]


def kernel(image_features, pos_table, masks):
    maskf = masks.astype(jnp.float32).reshape(M, 1, P)
    out = pl.pallas_call(
        _body,
        grid=(M,),
        in_specs=[
            pl.BlockSpec((1, 1, P), lambda m: (m, 0, 0)),
            pl.BlockSpec((P, D), lambda m: (0, 0)),
            pl.BlockSpec((P, D), lambda m: (0, 0)),
        ],
        out_specs=pl.BlockSpec((1, P, D), lambda m: (m, 0, 0)),
        out_shape=jax.ShapeDtypeStruct((M, P, D), jnp.float32),
    )(maskf, image_features, pos_table)
    return out
